# baseline (speedup 1.0000x reference)
import functools

import jax
import jax.numpy as jnp
from jax import lax
from jax.experimental import pallas as pl
from jax.experimental.pallas import tpu as pltpu
from jax.experimental.pallas import tpu_sc as plsc

N = 10000
E = 320000
D = 128
G = 16
NW = 32
FPT = D // NW
ECHUNK = 10000
NGRP = ECHUNK // 16
NCHUNK = E // ECHUNK
EPT_DEG = E // NW

_mesh = plsc.VectorSubcoreMesh(core_axis_name="c", subcore_axis_name="s")
_sc_params = pltpu.CompilerParams(needs_layout_passes=False)


@functools.partial(
    pl.kernel,
    mesh=_mesh,
    out_type=jax.ShapeDtypeStruct((NW, N), jnp.float32),
    scratch_types=[
        pltpu.VMEM((1, N), jnp.float32),
        pltpu.VMEM((EPT_DEG,), jnp.int32),
        pltpu.VMEM((EPT_DEG,), jnp.float32),
    ],
    compiler_params=_sc_params,
)
def _deg_kernel(col_hbm, w_hbm, out_hbm, deg_v, col_v, w_v):
    wid = lax.axis_index("s") * 2 + lax.axis_index("c")
    base = wid * EPT_DEG
    pltpu.sync_copy(col_hbm.at[pl.ds(base, EPT_DEG)], col_v)
    pltpu.sync_copy(w_hbm.at[pl.ds(base, EPT_DEG)], w_v)
    z16 = jnp.zeros((16,), jnp.float32)
    zi16 = jnp.zeros((16,), jnp.int32)

    @plsc.parallel_loop(0, N // 16, 1, unroll=4)
    def _(i):
        deg_v[0, pl.ds(i * 16, 16)] = z16

    @plsc.parallel_loop(0, EPT_DEG // 16, 1, unroll=4)
    def _(i):
        o = i * 16
        c16 = col_v[pl.ds(o, 16)]
        w16 = w_v[pl.ds(o, 16)]
        plsc.addupdate_scatter(deg_v, [zi16, c16], w16)

    pltpu.sync_copy(deg_v, out_hbm.at[pl.ds(wid, 1)])


@functools.partial(
    pl.kernel,
    mesh=_mesh,
    out_type=jax.ShapeDtypeStruct((D, N), jnp.float32),
    scratch_types=[
        pltpu.VMEM((2, N), jnp.int32),
        pltpu.VMEM((FPT, N), jnp.float32),
        pltpu.VMEM((ECHUNK,), jnp.int32),
        pltpu.VMEM((ECHUNK,), jnp.int32),
        pltpu.VMEM((ECHUNK,), jnp.float32),
        pltpu.VMEM((ECHUNK,), jnp.float32),
        pltpu.SemaphoreType.DMA,
        pltpu.SemaphoreType.DMA,
    ],
    compiler_params=_sc_params,
)
def _msg_kernel(yt_hbm, rc_hbm, w_hbm, out_hbm,
                y_v, acc_v, rc_v0, rc_v1, w_v0, w_v1, sem0, sem1):
    wid = lax.axis_index("s") * 2 + lax.axis_index("c")
    fbase = wid * FPT
    sems = (sem0, sem1)
    rc_bufs = (rc_v0, rc_v1)
    w_bufs = (w_v0, w_v1)

    def start_fetch(g, b):
        base = g * ECHUNK
        pltpu.async_copy(rc_hbm.at[pl.ds(base, ECHUNK)], rc_bufs[b], sems[b])
        pltpu.async_copy(w_hbm.at[pl.ds(base, ECHUNK)], w_bufs[b], sems[b])

    def wait_fetch(b):
        pltpu.make_async_copy(rc_hbm.at[pl.ds(0, ECHUNK)], rc_bufs[b], sems[b]).wait()
        pltpu.make_async_copy(w_hbm.at[pl.ds(0, ECHUNK)], w_bufs[b], sems[b]).wait()

    start_fetch(0, 0)
    start_fetch(1, 1)
    pltpu.sync_copy(yt_hbm.at[pl.ds(2 * wid, 2)], y_v)
    z16 = jnp.zeros((16,), jnp.float32)
    for f in range(FPT):
        @plsc.parallel_loop(0, N // 16, 1, unroll=4)
        def _(i, f=f):
            acc_v[f, pl.ds(i * 16, 16)] = z16

    f16s = [jnp.full((16,), f, jnp.int32) for f in range(FPT)]
    himask = jnp.full((16,), -65536, jnp.int32)

    def process(b):
        rb, wb = rc_bufs[b], w_bufs[b]

        @plsc.parallel_loop(0, NGRP, 1, unroll=4)
        def _(i):
            o = i * 16
            rc16 = rb[pl.ds(o, 16)]
            w16 = wb[pl.ds(o, 16)]
            r16 = jnp.bitwise_and(rc16, 0x3FFF)
            c16 = lax.shift_right_logical(rc16, 14)
            for p in range(2):
                vp = plsc.load_gather(y_v, [f16s[p], r16])
                va = plsc.bitcast(jnp.bitwise_and(vp, himask), jnp.float32)
                vb = plsc.bitcast(lax.shift_left(vp, 16), jnp.float32)
                plsc.addupdate_scatter(acc_v, [f16s[p], c16], w16 * va)
                plsc.addupdate_scatter(acc_v, [f16s[2 + p], c16], w16 * vb)

    def outer(j, c):
        for b in range(2):
            wait_fetch(b)
            process(b)
            start_fetch(j * 2 + b + 2, b)
        return c

    lax.fori_loop(0, (NCHUNK - 2) // 2, outer, 0)
    for b in range(2):
        wait_fetch(b)
        process(b)
    pltpu.sync_copy(acc_v.at[pl.ds(0, 2)], out_hbm.at[pl.ds(2 * wid, 2)])
    pltpu.sync_copy(acc_v.at[pl.ds(2, 2)], out_hbm.at[pl.ds(D // 2 + 2 * wid, 2)])


def _pack_pairs(y):
    au = lax.bitcast_convert_type(y[:D // 2], jnp.uint32)
    bu = lax.bitcast_convert_type(y[D // 2:], jnp.uint32)
    ar = jnp.bitwise_and(au + jnp.uint32(0x8000), jnp.uint32(0xFFFF0000))
    br = lax.shift_right_logical(bu + jnp.uint32(0x8000), jnp.uint32(16))
    return lax.bitcast_convert_type(jnp.bitwise_or(ar, br), jnp.int32)


def _tc_pre_body(deg_ref, x_ref, w_ref, y_ref, yp_ref, dis_ref):
    deg = jnp.sum(deg_ref[...], axis=0, keepdims=True) + 2.0
    dis = lax.rsqrt(deg)
    dis_ref[...] = dis
    xw = lax.dot_general(w_ref[...], x_ref[...], (((0,), (1,)), ((), ())),
                         preferred_element_type=jnp.float32)
    y = xw * dis
    y_ref[...] = y
    yp_ref[...] = _pack_pairs(y)


_tc_pre = pl.pallas_call(
    _tc_pre_body,
    out_shape=[jax.ShapeDtypeStruct((D, N), jnp.float32),
               jax.ShapeDtypeStruct((D // 2, N), jnp.int32),
               jax.ShapeDtypeStruct((1, N), jnp.float32)],
)


def _bn_leaky(s, y, dis, b2d, g2d, be2d):
    pre = dis * (s + 2.0 * y) + b2d
    mean = jnp.mean(pre, axis=1, keepdims=True)
    cent = pre - mean
    var = jnp.mean(cent * cent, axis=1, keepdims=True)
    hn = cent * lax.rsqrt(var + 1e-5) * g2d + be2d
    return jnp.where(hn >= 0, hn, 0.01 * hn)


def _tc_mid_body(s_ref, y_ref, dis_ref, b_ref, g_ref, be_ref, w_ref,
                 out_ref, outp_ref):
    dis = dis_ref[...]
    h = _bn_leaky(s_ref[...], y_ref[...], dis, b_ref[...], g_ref[...], be_ref[...])
    xw = lax.dot_general(w_ref[...], h, (((0,), (0,)), ((), ())),
                         preferred_element_type=jnp.float32)
    y = xw * dis
    out_ref[...] = y
    outp_ref[...] = _pack_pairs(y)


_tc_mid = pl.pallas_call(
    _tc_mid_body,
    out_shape=[jax.ShapeDtypeStruct((D, N), jnp.float32),
               jax.ShapeDtypeStruct((D // 2, N), jnp.int32)],
)


def _tc_fin_body(s_ref, y_ref, dis_ref, b_ref, g_ref, be_ref, batch_ref, out_ref):
    h = _bn_leaky(s_ref[...], y_ref[...], dis_ref[...], b_ref[...], g_ref[...],
                  be_ref[...])
    bvec = batch_ref[...]
    for gph in range(G):
        pen = jnp.where(bvec == gph, 0.0, -jnp.inf)
        out_ref[gph, :] = jnp.max(h + pen, axis=1)


_tc_fin = pl.pallas_call(
    _tc_fin_body,
    out_shape=jax.ShapeDtypeStruct((G, D), jnp.float32),
)


def kernel(x, edge_index, edge_attr, batch,
           W0, b0, g0, be0, W1, b1, g1, be1, W2, b2, g2, be2):
    row = edge_index[0].astype(jnp.int32)
    col = edge_index[1].astype(jnp.int32)
    w = edge_attr.astype(jnp.float32)
    batch2 = batch.astype(jnp.int32).reshape(1, N)
    rc = jnp.bitwise_or(row, lax.shift_left(col, 14))

    deg_parts = _deg_kernel(col, w)
    y, yp, dis = _tc_pre(deg_parts, x, W0)
    for l, (b_, g_, be_) in enumerate([(b0, g0, be0), (b1, g1, be1),
                                       (b2, g2, be2)]):
        s = _msg_kernel(yp, rc, w)
        b2d = b_.reshape(D, 1)
        g2d = g_.reshape(D, 1)
        be2d = be_.reshape(D, 1)
        if l == 0:
            y, yp = _tc_mid(s, y, dis, b2d, g2d, be2d, W1)
        elif l == 1:
            y, yp = _tc_mid(s, y, dis, b2d, g2d, be2d, W2)
        else:
            out = _tc_fin(s, y, dis, b2d, g2d, be2d, batch2)
    return out


# speedup vs baseline: 1.0174x; 1.0174x over previous
import functools

import jax
import jax.numpy as jnp
from jax import lax
from jax.experimental import pallas as pl
from jax.experimental.pallas import tpu as pltpu
from jax.experimental.pallas import tpu_sc as plsc

N = 10000
E = 320000
D = 128
G = 16
NW = 32
FPT = D // NW
ECHUNK = 8000
NGRP = ECHUNK // 16
NCHUNK = E // ECHUNK
EPT_DEG = E // NW

_mesh = plsc.VectorSubcoreMesh(core_axis_name="c", subcore_axis_name="s")
_sc_params = pltpu.CompilerParams(needs_layout_passes=False)


@functools.partial(
    pl.kernel,
    mesh=_mesh,
    out_type=jax.ShapeDtypeStruct((NW, N), jnp.float32),
    scratch_types=[
        pltpu.VMEM((1, N), jnp.float32),
        pltpu.VMEM((EPT_DEG,), jnp.int32),
        pltpu.VMEM((EPT_DEG,), jnp.float32),
    ],
    compiler_params=_sc_params,
)
def _deg_kernel(col_hbm, w_hbm, out_hbm, deg_v, col_v, w_v):
    wid = lax.axis_index("s") * 2 + lax.axis_index("c")
    base = wid * EPT_DEG
    pltpu.sync_copy(col_hbm.at[pl.ds(base, EPT_DEG)], col_v)
    pltpu.sync_copy(w_hbm.at[pl.ds(base, EPT_DEG)], w_v)
    z16 = jnp.zeros((16,), jnp.float32)
    zi16 = jnp.zeros((16,), jnp.int32)

    @plsc.parallel_loop(0, N // 16, 1, unroll=4)
    def _(i):
        deg_v[0, pl.ds(i * 16, 16)] = z16

    @plsc.parallel_loop(0, EPT_DEG // 16, 1, unroll=4)
    def _(i):
        o = i * 16
        c16 = col_v[pl.ds(o, 16)]
        w16 = w_v[pl.ds(o, 16)]
        plsc.addupdate_scatter(deg_v, [zi16, c16], w16)

    pltpu.sync_copy(deg_v, out_hbm.at[pl.ds(wid, 1)])


@functools.partial(
    pl.kernel,
    mesh=_mesh,
    out_type=jax.ShapeDtypeStruct((D, N), jnp.float32),
    scratch_types=[
        pltpu.VMEM((2, N), jnp.int32),
        pltpu.VMEM((FPT, N), jnp.float32),
        pltpu.VMEM((ECHUNK,), jnp.int32),
        pltpu.VMEM((ECHUNK,), jnp.int32),
        pltpu.VMEM((ECHUNK,), jnp.float32),
        pltpu.VMEM((ECHUNK,), jnp.float32),
        pltpu.SemaphoreType.DMA,
        pltpu.SemaphoreType.DMA,
    ],
    compiler_params=_sc_params,
)
def _msg_kernel(yt_hbm, rc_hbm, w_hbm, out_hbm,
                y_v, acc_v, rc_v0, rc_v1, w_v0, w_v1, sem0, sem1):
    wid = lax.axis_index("s") * 2 + lax.axis_index("c")
    fbase = wid * FPT
    sems = (sem0, sem1)
    rc_bufs = (rc_v0, rc_v1)
    w_bufs = (w_v0, w_v1)

    def start_fetch(g, b):
        base = g * ECHUNK
        pltpu.async_copy(rc_hbm.at[pl.ds(base, ECHUNK)], rc_bufs[b], sems[b])
        pltpu.async_copy(w_hbm.at[pl.ds(base, ECHUNK)], w_bufs[b], sems[b])

    def wait_fetch(b):
        pltpu.make_async_copy(rc_hbm.at[pl.ds(0, ECHUNK)], rc_bufs[b], sems[b]).wait()
        pltpu.make_async_copy(w_hbm.at[pl.ds(0, ECHUNK)], w_bufs[b], sems[b]).wait()

    start_fetch(0, 0)
    start_fetch(1, 1)
    pltpu.sync_copy(yt_hbm.at[pl.ds(2 * wid, 2)], y_v)
    z16 = jnp.zeros((16,), jnp.float32)
    for f in range(FPT):
        @plsc.parallel_loop(0, N // 16, 1, unroll=4)
        def _(i, f=f):
            acc_v[f, pl.ds(i * 16, 16)] = z16

    f16s = [jnp.full((16,), f, jnp.int32) for f in range(FPT)]
    himask = jnp.full((16,), -65536, jnp.int32)

    def process(b):
        rb, wb = rc_bufs[b], w_bufs[b]

        @plsc.parallel_loop(0, NGRP, 1, unroll=2)
        def _(i):
            o = i * 16
            rc16 = rb[pl.ds(o, 16)]
            w16 = wb[pl.ds(o, 16)]
            r16 = jnp.bitwise_and(rc16, 0x3FFF)
            c16 = lax.shift_right_logical(rc16, 14)
            for p in range(2):
                vp = plsc.load_gather(y_v, [f16s[p], r16])
                va = plsc.bitcast(jnp.bitwise_and(vp, himask), jnp.float32)
                vb = plsc.bitcast(lax.shift_left(vp, 16), jnp.float32)
                plsc.addupdate_scatter(acc_v, [f16s[p], c16], w16 * va)
                plsc.addupdate_scatter(acc_v, [f16s[2 + p], c16], w16 * vb)

    def outer(j, c):
        for b in range(2):
            wait_fetch(b)
            process(b)
            start_fetch(j * 2 + b + 2, b)
        return c

    lax.fori_loop(0, (NCHUNK - 2) // 2, outer, 0)
    for b in range(2):
        wait_fetch(b)
        process(b)
    pltpu.sync_copy(acc_v.at[pl.ds(0, 2)], out_hbm.at[pl.ds(2 * wid, 2)])
    pltpu.sync_copy(acc_v.at[pl.ds(2, 2)], out_hbm.at[pl.ds(D // 2 + 2 * wid, 2)])


def _pack_pairs(y):
    au = lax.bitcast_convert_type(y[:D // 2], jnp.uint32)
    bu = lax.bitcast_convert_type(y[D // 2:], jnp.uint32)
    ar = jnp.bitwise_and(au + jnp.uint32(0x8000), jnp.uint32(0xFFFF0000))
    br = lax.shift_right_logical(bu + jnp.uint32(0x8000), jnp.uint32(16))
    return lax.bitcast_convert_type(jnp.bitwise_or(ar, br), jnp.int32)


def _tc_pre_body(deg_ref, x_ref, w_ref, y_ref, yp_ref, dis_ref):
    deg = jnp.sum(deg_ref[...], axis=0, keepdims=True) + 2.0
    dis = lax.rsqrt(deg)
    dis_ref[...] = dis
    xw = lax.dot_general(w_ref[...], x_ref[...], (((0,), (1,)), ((), ())),
                         preferred_element_type=jnp.float32)
    y = xw * dis
    y_ref[...] = y
    yp_ref[...] = _pack_pairs(y)


_tc_pre = pl.pallas_call(
    _tc_pre_body,
    out_shape=[jax.ShapeDtypeStruct((D, N), jnp.float32),
               jax.ShapeDtypeStruct((D // 2, N), jnp.int32),
               jax.ShapeDtypeStruct((1, N), jnp.float32)],
)


def _bn_leaky(s, y, dis, b2d, g2d, be2d):
    pre = dis * (s + 2.0 * y) + b2d
    mean = jnp.mean(pre, axis=1, keepdims=True)
    cent = pre - mean
    var = jnp.mean(cent * cent, axis=1, keepdims=True)
    hn = cent * lax.rsqrt(var + 1e-5) * g2d + be2d
    return jnp.where(hn >= 0, hn, 0.01 * hn)


def _tc_mid_body(s_ref, y_ref, dis_ref, b_ref, g_ref, be_ref, w_ref,
                 out_ref, outp_ref):
    dis = dis_ref[...]
    h = _bn_leaky(s_ref[...], y_ref[...], dis, b_ref[...], g_ref[...], be_ref[...])
    xw = lax.dot_general(w_ref[...], h, (((0,), (0,)), ((), ())),
                         preferred_element_type=jnp.float32)
    y = xw * dis
    out_ref[...] = y
    outp_ref[...] = _pack_pairs(y)


_tc_mid = pl.pallas_call(
    _tc_mid_body,
    out_shape=[jax.ShapeDtypeStruct((D, N), jnp.float32),
               jax.ShapeDtypeStruct((D // 2, N), jnp.int32)],
)


def _tc_fin_body(s_ref, y_ref, dis_ref, b_ref, g_ref, be_ref, batch_ref, out_ref):
    h = _bn_leaky(s_ref[...], y_ref[...], dis_ref[...], b_ref[...], g_ref[...],
                  be_ref[...])
    bvec = batch_ref[...]
    for gph in range(G):
        pen = jnp.where(bvec == gph, 0.0, -jnp.inf)
        out_ref[gph, :] = jnp.max(h + pen, axis=1)


_tc_fin = pl.pallas_call(
    _tc_fin_body,
    out_shape=jax.ShapeDtypeStruct((G, D), jnp.float32),
)


def kernel(x, edge_index, edge_attr, batch,
           W0, b0, g0, be0, W1, b1, g1, be1, W2, b2, g2, be2):
    row = edge_index[0].astype(jnp.int32)
    col = edge_index[1].astype(jnp.int32)
    w = edge_attr.astype(jnp.float32)
    batch2 = batch.astype(jnp.int32).reshape(1, N)
    rc = jnp.bitwise_or(row, lax.shift_left(col, 14))

    deg_parts = _deg_kernel(col, w)
    y, yp, dis = _tc_pre(deg_parts, x, W0)
    for l, (b_, g_, be_) in enumerate([(b0, g0, be0), (b1, g1, be1),
                                       (b2, g2, be2)]):
        s = _msg_kernel(yp, rc, w)
        b2d = b_.reshape(D, 1)
        g2d = g_.reshape(D, 1)
        be2d = be_.reshape(D, 1)
        if l == 0:
            y, yp = _tc_mid(s, y, dis, b2d, g2d, be2d, W1)
        elif l == 1:
            y, yp = _tc_mid(s, y, dis, b2d, g2d, be2d, W2)
        else:
            out = _tc_fin(s, y, dis, b2d, g2d, be2d, batch2)
    return out


# speedup vs baseline: 1.0284x; 1.0109x over previous
import functools

import jax
import jax.numpy as jnp
from jax import lax
from jax.experimental import pallas as pl
from jax.experimental.pallas import tpu as pltpu
from jax.experimental.pallas import tpu_sc as plsc

N = 10000
E = 320000
D = 128
G = 16
NW = 32
FPT = D // NW
ECHUNK = 8000
NGRP = ECHUNK // 16
NCHUNK = E // ECHUNK
EPT_DEG = E // NW

_mesh = plsc.VectorSubcoreMesh(core_axis_name="c", subcore_axis_name="s")
_sc_params = pltpu.CompilerParams(needs_layout_passes=False)


@functools.partial(
    pl.kernel,
    mesh=_mesh,
    out_type=jax.ShapeDtypeStruct((NW, N), jnp.float32),
    scratch_types=[
        pltpu.VMEM((1, N), jnp.float32),
        pltpu.VMEM((EPT_DEG,), jnp.int32),
        pltpu.VMEM((EPT_DEG,), jnp.float32),
    ],
    compiler_params=_sc_params,
)
def _deg_kernel(col_hbm, w_hbm, out_hbm, deg_v, col_v, w_v):
    wid = lax.axis_index("s") * 2 + lax.axis_index("c")
    base = wid * EPT_DEG
    pltpu.sync_copy(col_hbm.at[pl.ds(base, EPT_DEG)], col_v)
    pltpu.sync_copy(w_hbm.at[pl.ds(base, EPT_DEG)], w_v)
    z16 = jnp.zeros((16,), jnp.float32)
    zi16 = jnp.zeros((16,), jnp.int32)

    @plsc.parallel_loop(0, N // 16, 1, unroll=4)
    def _(i):
        deg_v[0, pl.ds(i * 16, 16)] = z16

    @plsc.parallel_loop(0, EPT_DEG // 16, 1, unroll=4)
    def _(i):
        o = i * 16
        c16 = col_v[pl.ds(o, 16)]
        w16 = w_v[pl.ds(o, 16)]
        plsc.addupdate_scatter(deg_v, [zi16, c16], w16)

    pltpu.sync_copy(deg_v, out_hbm.at[pl.ds(wid, 1)])


@functools.partial(
    pl.kernel,
    mesh=_mesh,
    out_type=jax.ShapeDtypeStruct((D, N), jnp.float32),
    scratch_types=[
        pltpu.VMEM((2, N), jnp.int32),
        pltpu.VMEM((FPT, N), jnp.float32),
        pltpu.VMEM((ECHUNK,), jnp.int32),
        pltpu.VMEM((ECHUNK,), jnp.int32),
        pltpu.VMEM((ECHUNK,), jnp.float32),
        pltpu.VMEM((ECHUNK,), jnp.float32),
        pltpu.SemaphoreType.DMA,
        pltpu.SemaphoreType.DMA,
    ],
    compiler_params=_sc_params,
)
def _msg_kernel(yt_hbm, rc_hbm, w_hbm, out_hbm,
                y_v, acc_v, rc_v0, rc_v1, w_v0, w_v1, sem0, sem1):
    wid = lax.axis_index("s") * 2 + lax.axis_index("c")
    fbase = wid * FPT
    sems = (sem0, sem1)
    rc_bufs = (rc_v0, rc_v1)
    w_bufs = (w_v0, w_v1)

    def start_fetch(g, b):
        base = g * ECHUNK
        pltpu.async_copy(rc_hbm.at[pl.ds(base, ECHUNK)], rc_bufs[b], sems[b])
        pltpu.async_copy(w_hbm.at[pl.ds(base, ECHUNK)], w_bufs[b], sems[b])

    def wait_fetch(b):
        pltpu.make_async_copy(rc_hbm.at[pl.ds(0, ECHUNK)], rc_bufs[b], sems[b]).wait()
        pltpu.make_async_copy(w_hbm.at[pl.ds(0, ECHUNK)], w_bufs[b], sems[b]).wait()

    start_fetch(0, 0)
    start_fetch(1, 1)
    pltpu.sync_copy(yt_hbm.at[pl.ds(2 * wid, 2)], y_v)
    z16 = jnp.zeros((16,), jnp.float32)
    for f in range(FPT):
        @plsc.parallel_loop(0, N // 16, 1, unroll=4)
        def _(i, f=f):
            acc_v[f, pl.ds(i * 16, 16)] = z16

    f16s = [jnp.full((16,), f, jnp.int32) for f in range(FPT)]
    himask = jnp.full((16,), -65536, jnp.int32)

    def process(b):
        rb, wb = rc_bufs[b], w_bufs[b]

        @plsc.parallel_loop(0, NGRP, 1, unroll=2)
        def _(i):
            o = i * 16
            rc16 = rb[pl.ds(o, 16)]
            w16 = wb[pl.ds(o, 16)]
            r16 = jnp.bitwise_and(rc16, 0x3FFF)
            c16 = lax.shift_right_logical(rc16, 14)
            for p in range(2):
                vp = plsc.load_gather(y_v, [f16s[p], r16])
                va = plsc.bitcast(jnp.bitwise_and(vp, himask), jnp.float32)
                vb = plsc.bitcast(lax.shift_left(vp, 16), jnp.float32)
                plsc.addupdate_scatter(acc_v, [f16s[p], c16], w16 * va)
                plsc.addupdate_scatter(acc_v, [f16s[2 + p], c16], w16 * vb)

    def outer(j, c):
        for b in range(2):
            wait_fetch(b)
            process(b)
            start_fetch(j * 2 + b + 2, b)
        return c

    lax.fori_loop(0, (NCHUNK - 2) // 2, outer, 0)
    for b in range(2):
        wait_fetch(b)
        process(b)
    pltpu.sync_copy(acc_v.at[pl.ds(0, 2)], out_hbm.at[pl.ds(2 * wid, 2)])
    pltpu.sync_copy(acc_v.at[pl.ds(2, 2)], out_hbm.at[pl.ds(D // 2 + 2 * wid, 2)])


def _pack_pairs(y):
    au = lax.bitcast_convert_type(y[:D // 2], jnp.uint32)
    bu = lax.bitcast_convert_type(y[D // 2:], jnp.uint32)
    ar = jnp.bitwise_and(au + jnp.uint32(0x8000), jnp.uint32(0xFFFF0000))
    br = lax.shift_right_logical(bu + jnp.uint32(0x8000), jnp.uint32(16))
    return lax.bitcast_convert_type(jnp.bitwise_or(ar, br), jnp.int32)


def _unpack_pairs(yp):
    u = lax.bitcast_convert_type(yp, jnp.uint32)
    a = jnp.bitwise_and(u, jnp.uint32(0xFFFF0000))
    b = lax.shift_left(u, jnp.uint32(16))
    return lax.bitcast_convert_type(jnp.concatenate([a, b], axis=0),
                                    jnp.float32)


def _tc_pre_body(deg_ref, x_ref, w_ref, yp_ref, dis_ref):
    deg = jnp.sum(deg_ref[...], axis=0, keepdims=True) + 2.0
    dis = lax.rsqrt(deg)
    dis_ref[...] = dis
    xw = lax.dot_general(w_ref[...], x_ref[...], (((0,), (1,)), ((), ())),
                         preferred_element_type=jnp.float32)
    yp_ref[...] = _pack_pairs(xw * dis)


_tc_pre = pl.pallas_call(
    _tc_pre_body,
    out_shape=[jax.ShapeDtypeStruct((D // 2, N), jnp.int32),
               jax.ShapeDtypeStruct((1, N), jnp.float32)],
)


def _bn_leaky(s, y, dis, b2d, g2d, be2d):
    pre = dis * (s + 2.0 * y) + b2d
    mean = jnp.mean(pre, axis=1, keepdims=True)
    cent = pre - mean
    var = jnp.mean(cent * cent, axis=1, keepdims=True)
    hn = cent * lax.rsqrt(var + 1e-5) * g2d + be2d
    return jnp.where(hn >= 0, hn, 0.01 * hn)


def _tc_mid_body(s_ref, yp_ref, dis_ref, b_ref, g_ref, be_ref, w_ref,
                 outp_ref):
    dis = dis_ref[...]
    y = _unpack_pairs(yp_ref[...])
    h = _bn_leaky(s_ref[...], y, dis, b_ref[...], g_ref[...], be_ref[...])
    xw = lax.dot_general(w_ref[...], h, (((0,), (0,)), ((), ())),
                         preferred_element_type=jnp.float32)
    outp_ref[...] = _pack_pairs(xw * dis)


_tc_mid = pl.pallas_call(
    _tc_mid_body,
    out_shape=jax.ShapeDtypeStruct((D // 2, N), jnp.int32),
)


def _tc_fin_body(s_ref, yp_ref, dis_ref, b_ref, g_ref, be_ref, batch_ref, out_ref):
    h = _bn_leaky(s_ref[...], _unpack_pairs(yp_ref[...]), dis_ref[...],
                  b_ref[...], g_ref[...], be_ref[...])
    bvec = batch_ref[...]
    for gph in range(G):
        pen = jnp.where(bvec == gph, 0.0, -jnp.inf)
        out_ref[gph, :] = jnp.max(h + pen, axis=1)


_tc_fin = pl.pallas_call(
    _tc_fin_body,
    out_shape=jax.ShapeDtypeStruct((G, D), jnp.float32),
)


def kernel(x, edge_index, edge_attr, batch,
           W0, b0, g0, be0, W1, b1, g1, be1, W2, b2, g2, be2):
    row = edge_index[0].astype(jnp.int32)
    col = edge_index[1].astype(jnp.int32)
    w = edge_attr.astype(jnp.float32)
    batch2 = batch.astype(jnp.int32).reshape(1, N)
    rc = jnp.bitwise_or(row, lax.shift_left(col, 14))

    deg_parts = _deg_kernel(col, w)
    yp, dis = _tc_pre(deg_parts, x, W0)
    for l, (b_, g_, be_) in enumerate([(b0, g0, be0), (b1, g1, be1),
                                       (b2, g2, be2)]):
        s = _msg_kernel(yp, rc, w)
        b2d = b_.reshape(D, 1)
        g2d = g_.reshape(D, 1)
        be2d = be_.reshape(D, 1)
        if l == 0:
            yp = _tc_mid(s, yp, dis, b2d, g2d, be2d, W1)
        elif l == 1:
            yp = _tc_mid(s, yp, dis, b2d, g2d, be2d, W2)
        else:
            out = _tc_fin(s, yp, dis, b2d, g2d, be2d, batch2)
    return out
